# Initial kernel scaffold; baseline (speedup 1.0000x reference)
import jax
import jax.numpy as jnp
from jax.experimental import pallas as pl


def kernel(pv_y_osgb_fourier, pv_x_osgb_fourier, pv_x_osgb, pv, pv_time_utc_fourier, solar_azimuth, solar_elevation, pv_system_row_number, embedding_table, start_idx_5_min):
    raise NotImplementedError("write your pallas kernel here")


# speedup vs baseline: 1.5585x; 1.5585x over previous
import functools

import jax
import jax.numpy as jnp
from jax import lax
from jax.experimental import pallas as pl
from jax.experimental.pallas import tpu as pltpu
from jax.experimental.pallas import tpu_sc as plsc

B = 4096
NPV = 200
HALF = NPV // 2
B2 = B * 2
EMB = 16
OUTF = 42
NW = 32
NB = 8
NBR = NB * 2
RPW = B2 // NW
CPW = RPW // NBR


def _sc_body(y_hbm, x_hbm, idx_hbm, tf_hbm, az_hbm, el_hbm, tab_hbm, out_hbm,
             idx_v, out_v, emb_v, cb_v, gsem):
    wid = lax.axis_index("s") * 2 + lax.axis_index("c")
    iota = lax.iota(jnp.int32, 16)

    def chunk(g, carry):
        r0 = wid * RPW + g * NBR
        b0 = wid * (RPW // 2) + g * NB
        pltpu.sync_copy(idx_hbm.at[pl.ds(r0, NBR)], idx_v)
        pltpu.sync_copy(y_hbm.at[pl.ds(r0, NBR)], out_v.at[:, :, pl.ds(0, 8)])
        pltpu.sync_copy(x_hbm.at[pl.ds(r0, NBR)], out_v.at[:, :, pl.ds(8, 8)])
        pltpu.sync_copy(tf_hbm.at[pl.ds(b0 * 8, NB * 8)], cb_v.at[pl.ds(0, NB * 8)])
        pltpu.sync_copy(az_hbm.at[pl.ds(b0, NB)], cb_v.at[pl.ds(NB * 8, NB)])
        pltpu.sync_copy(el_hbm.at[pl.ds(b0, NB)], cb_v.at[pl.ds(NB * 9, NB)])
        descs = []
        for j in range(NBR):
            descs.append(pltpu.async_copy(tab_hbm.at[idx_v.at[j]], emb_v.at[j], gsem))
        for k in range(NB):
            gidx = jnp.where(iota < 8, k * 8 + iota,
                             jnp.where(iota == 8, NB * 8 + k,
                                       jnp.where(iota == 9, NB * 9 + k, NB * 10)))
            bvec = plsc.load_gather(cb_v, [gidx])
            for j2 in (2 * k, 2 * k + 1):
                def vrow(m, c, _j2=j2, _bvec=bvec):
                    for dn in range(5):
                        out_v[_j2, m * 5 + dn, pl.ds(16, 16)] = _bvec
                    return c
                lax.fori_loop(0, HALF // 5, vrow, 0)
        for d in descs:
            d.wait()
        iota26 = iota + 26
        for j2 in range(NBR):
            jvec = jnp.full((16,), j2, jnp.int32)

            def erow(m, c, _j2=j2, _jvec=jvec):
                for dn in range(5):
                    n = m * 5 + dn
                    ev = emb_v[_j2, n, :]
                    plsc.store_scatter(
                        out_v, [_jvec, jnp.full((16,), n, jnp.int32), iota26], ev)
                return c
            lax.fori_loop(0, HALF // 5, erow, 0)
        pltpu.sync_copy(out_v, out_hbm.at[pl.ds(r0, NBR)])
        return carry

    lax.fori_loop(0, CPW, chunk, 0)


_sc_call = functools.partial(
    pl.kernel,
    out_type=jax.ShapeDtypeStruct((B2, HALF, OUTF), jnp.float32),
    mesh=plsc.VectorSubcoreMesh(core_axis_name="c", subcore_axis_name="s"),
    compiler_params=pltpu.CompilerParams(
        needs_layout_passes=False, use_tc_tiling_on_sc=False),
    scratch_types=[
        pltpu.VMEM((NBR, HALF), jnp.int32),
        pltpu.VMEM((NBR, HALF, OUTF), jnp.float32),
        pltpu.VMEM((NBR, HALF, EMB), jnp.float32),
        pltpu.VMEM((NB * 10 + 16, ), jnp.float32),
        pltpu.SemaphoreType.DMA,
    ],
)(_sc_body)


def kernel(pv_y_osgb_fourier, pv_x_osgb_fourier, pv_x_osgb, pv,
           pv_time_utc_fourier, solar_azimuth, solar_elevation,
           pv_system_row_number, embedding_table, start_idx_5_min=0):
    t = 6 + start_idx_5_min
    tf6 = lax.dynamic_slice_in_dim(pv_time_utc_fourier, t, 1, axis=1)[:, 0, :]
    az6 = lax.dynamic_slice_in_dim(solar_azimuth, t, 1, axis=1)[:, 0]
    el6 = lax.dynamic_slice_in_dim(solar_elevation, t, 1, axis=1)[:, 0]
    y2 = pv_y_osgb_fourier.reshape(B2, HALF, 8)
    x2 = pv_x_osgb_fourier.reshape(B2, HALF, 8)
    idx2 = pv_system_row_number.astype(jnp.int32).reshape(B2, HALF)
    out = _sc_call(y2, x2, idx2, tf6.reshape(B * 8), az6, el6, embedding_table)
    return out.reshape(B, NPV, OUTF)


# speedup vs baseline: 7.6065x; 4.8807x over previous
import functools

import jax
import jax.numpy as jnp
from jax import lax
from jax.experimental import pallas as pl
from jax.experimental.pallas import tpu as pltpu
from jax.experimental.pallas import tpu_sc as plsc

B = 4096
NPV = 200
EMB = 16
OUTF = 42
NT = NPV // 8
BT = B // 128
NGRP = NT


def _sc_body(y_hbm, x_hbm, idx_hbm, tf_hbm, az_hbm, el_hbm, tab_hbm, out_hbm,
             idx_v, emb_v, embT_v, bc_v, tf_v, az_v, el_v, yx_v,
             gsem, esem0, wsem):
    bt = lax.axis_index("s") * 2 + lax.axis_index("c")
    iota = lax.iota(jnp.int32, 16)
    boff = pl.multiple_of(bt * 128, 128)

    pltpu.sync_copy(idx_hbm.at[:, bt], idx_v)
    pltpu.sync_copy(tf_hbm.at[:, pl.ds(boff, 128)], tf_v)
    pltpu.sync_copy(az_hbm.at[pl.ds(boff, 128)], az_v)
    pltpu.sync_copy(el_hbm.at[pl.ds(boff, 128)], el_v)

    for f in range(8):
        pltpu.sync_copy(y_hbm.at[:, :, bt, f, :], yx_v)
        pltpu.sync_copy(yx_v, out_hbm.at[f, :, bt, :, :])
        pltpu.sync_copy(x_hbm.at[:, :, bt, f, :], yx_v)
        pltpu.sync_copy(yx_v, out_hbm.at[8 + f, :, bt, :, :])

    for cc in range(10):
        for i in range(8):
            if cc < 8:
                seg = tf_v[cc, pl.ds(i * 16, 16)]
            elif cc == 8:
                seg = az_v[pl.ds(i * 16, 16)]
            else:
                seg = el_v[pl.ds(i * 16, 16)]

            def brow(r, carry, _cc=cc, _i=i, _seg=seg):
                bc_v[_cc, r, pl.ds(_i * 16, 16)] = _seg
                return carry
            lax.fori_loop(0, 8, brow, 0)

    def bc_fire(j, carry):
        bds = [pltpu.async_copy(bc_v.at[cc], out_hbm.at[16 + cc, j, bt], wsem)
               for cc in range(10)]
        for d in bds:
            d.wait()
        return carry
    lax.fori_loop(0, NT, bc_fire, 0)

    def grp(j, carry):
        gds = []
        for j2 in range(8):
            gds.append(pltpu.async_copy(tab_hbm.at[idx_v.at[j, j2]],
                                        emb_v.at[j2], gsem))
        for d in gds:
            d.wait()

        def tr(r2, carry2):
            for e in range(EMB):
                for i in range(8):
                    vec = plsc.load_gather(
                        emb_v, [jnp.full((16,), r2, jnp.int32),
                                i * 16 + iota,
                                jnp.full((16,), e, jnp.int32)])
                    embT_v[e, r2, pl.ds(i * 16, 16)] = vec
            return carry2
        lax.fori_loop(0, 8, tr, 0)

        wds = []
        for e in range(EMB):
            wds.append(pltpu.async_copy(embT_v.at[e],
                                        out_hbm.at[26 + e, j, bt], esem0))
        for d in wds:
            d.wait()
        return carry
    lax.fori_loop(0, NGRP, grp, 0)


_sc_call = functools.partial(
    pl.kernel,
    out_type=jax.ShapeDtypeStruct((OUTF, NT, BT, 8, 128), jnp.float32),
    mesh=plsc.VectorSubcoreMesh(core_axis_name="c", subcore_axis_name="s"),
    compiler_params=pltpu.CompilerParams(
        needs_layout_passes=False, use_tc_tiling_on_sc=False),
    scratch_types=[
        pltpu.VMEM((NT, 8, 128), jnp.int32),
        pltpu.VMEM((8, 128, EMB), jnp.float32),
        pltpu.VMEM((EMB, 8, 128), jnp.float32),
        pltpu.VMEM((10, 8, 128), jnp.float32),
        pltpu.VMEM((8, 128), jnp.float32),
        pltpu.VMEM((128,), jnp.float32),
        pltpu.VMEM((128,), jnp.float32),
        pltpu.VMEM((NT, 8, 128), jnp.float32),
        pltpu.SemaphoreType.DMA,
        pltpu.SemaphoreType.DMA,
        pltpu.SemaphoreType.DMA,
    ],
)(_sc_body)


def kernel(pv_y_osgb_fourier, pv_x_osgb_fourier, pv_x_osgb, pv,
           pv_time_utc_fourier, solar_azimuth, solar_elevation,
           pv_system_row_number, embedding_table, start_idx_5_min=0):
    t = 6 + start_idx_5_min
    tf6 = lax.dynamic_slice_in_dim(pv_time_utc_fourier, t, 1, axis=1)[:, 0, :]
    az6 = lax.dynamic_slice_in_dim(solar_azimuth, t, 1, axis=1)[:, 0]
    el6 = lax.dynamic_slice_in_dim(solar_elevation, t, 1, axis=1)[:, 0]
    idx = pv_system_row_number.astype(jnp.int32)

    y5 = (pv_y_osgb_fourier.transpose(1, 2, 0).reshape(NPV, 8, BT, 128)
          .transpose(0, 2, 1, 3).reshape(NT, 8, BT, 8, 128))
    x5 = (pv_x_osgb_fourier.transpose(1, 2, 0).reshape(NPV, 8, BT, 128)
          .transpose(0, 2, 1, 3).reshape(NT, 8, BT, 8, 128))
    idx5 = (idx.transpose(1, 0).reshape(NT, 8, BT, 128)
            .transpose(0, 2, 1, 3))
    out_nat = _sc_call(y5, x5, idx5, tf6.transpose(1, 0), az6, el6,
                       embedding_table)
    return out_nat.transpose(2, 4, 1, 3, 0).reshape(B, NPV, OUTF)


# speedup vs baseline: 8.1450x; 1.0708x over previous
import functools

import jax
import jax.numpy as jnp
from jax import lax
from jax.experimental import pallas as pl
from jax.experimental.pallas import tpu as pltpu
from jax.experimental.pallas import tpu_sc as plsc

B = 4096
NPV = 200
EMB = 16
OUTF = 42
NT = NPV // 8
BT = B // 128
NGRP = NT


def _sc_body(y_hbm, x_hbm, idx_hbm, tf_hbm, az_hbm, el_hbm, tab_hbm, out_hbm,
             idx_v, emb_v, embT_v, bc_v, tf_v, az_v, el_v, yx_v,
             gsem, esem0, wsem, ysem):
    bt = lax.axis_index("s") * 2 + lax.axis_index("c")
    iota = lax.iota(jnp.int32, 16)
    boff = pl.multiple_of(bt * 128, 128)

    pltpu.sync_copy(tf_hbm.at[:, pl.ds(boff, 128)], tf_v)
    pltpu.sync_copy(az_hbm.at[pl.ds(boff, 128)], az_v)
    pltpu.sync_copy(el_hbm.at[pl.ds(boff, 128)], el_v)

    def yx_src(f):
        h = y_hbm if f < 8 else x_hbm
        return h.at[:, :, bt, f % 8, :]

    rds, wds_yx = {}, {}
    rds[0] = pltpu.async_copy(yx_src(0), yx_v.at[0], ysem)
    for f in range(16):
        b = f % 2
        rds[f].wait()
        wds_yx[f] = pltpu.async_copy(yx_v.at[b], out_hbm.at[f, :, bt], wsem)
        if f + 1 < 16:
            if f >= 1:
                wds_yx[f - 1].wait()
            rds[f + 1] = pltpu.async_copy(yx_src(f + 1), yx_v.at[1 - b], ysem)
    wds_yx[14].wait()
    wds_yx[15].wait()

    for cc in range(10):
        for i in range(8):
            if cc < 8:
                seg = tf_v[cc, pl.ds(i * 16, 16)]
            elif cc == 8:
                seg = az_v[pl.ds(i * 16, 16)]
            else:
                seg = el_v[pl.ds(i * 16, 16)]

            def brow(r, carry, _cc=cc, _i=i, _seg=seg):
                bc_v[_cc, r, pl.ds(_i * 16, 16)] = _seg
                return carry
            lax.fori_loop(0, 8, brow, 0)

    def idx_stage(j, gb):
        pltpu.sync_copy(idx_hbm.at[j, bt], idx_v.at[gb])

    def g_fire(gb):
        for j2 in range(8):
            pltpu.async_copy(tab_hbm.at[idx_v.at[gb, j2]],
                             emb_v.at[gb, j2], gsem)

    def g_drain(gb):
        for j2 in range(8):
            pltpu.make_async_copy(tab_hbm.at[idx_v.at[gb, j2]],
                                  emb_v.at[gb, j2], gsem).wait()

    def transpose(gb):
        def tr(r2, carry2):
            for e in range(EMB):
                for i in range(8):
                    vec = plsc.load_gather(
                        emb_v, [jnp.full((16,), gb, jnp.int32),
                                jnp.full((16,), r2, jnp.int32),
                                i * 16 + iota,
                                jnp.full((16,), e, jnp.int32)])
                    embT_v[e, r2, pl.ds(i * 16, 16)] = vec
            return carry2
        lax.fori_loop(0, 8, tr, 0)

    def sub_iter(j, gb, fire_next):
        bds = [pltpu.async_copy(bc_v.at[cc], out_hbm.at[16 + cc, j, bt], wsem)
               for cc in range(10)]
        g_drain(gb)
        if fire_next:
            idx_stage(j + 1, 1 - gb)
            g_fire(1 - gb)
        transpose(gb)
        eds = [pltpu.async_copy(embT_v.at[e], out_hbm.at[26 + e, j, bt], esem0)
               for e in range(EMB)]
        for d in bds + eds:
            d.wait()

    idx_stage(0, 0)
    g_fire(0)

    def pipe(t, carry):
        sub_iter(2 * t, 0, True)
        sub_iter(2 * t + 1, 1, True)
        return carry
    lax.fori_loop(0, (NGRP - 1) // 2, pipe, 0)
    sub_iter(NGRP - 1, 0, False)


_sc_call = functools.partial(
    pl.kernel,
    out_type=jax.ShapeDtypeStruct((OUTF, NT, BT, 8, 128), jnp.float32),
    mesh=plsc.VectorSubcoreMesh(core_axis_name="c", subcore_axis_name="s"),
    compiler_params=pltpu.CompilerParams(
        needs_layout_passes=False, use_tc_tiling_on_sc=False),
    scratch_types=[
        pltpu.VMEM((2, 8, 128), jnp.int32),
        pltpu.VMEM((2, 8, 128, EMB), jnp.float32),
        pltpu.VMEM((EMB, 8, 128), jnp.float32),
        pltpu.VMEM((10, 8, 128), jnp.float32),
        pltpu.VMEM((8, 128), jnp.float32),
        pltpu.VMEM((128,), jnp.float32),
        pltpu.VMEM((128,), jnp.float32),
        pltpu.VMEM((2, NT, 8, 128), jnp.float32),
        pltpu.SemaphoreType.DMA,
        pltpu.SemaphoreType.DMA,
        pltpu.SemaphoreType.DMA,
        pltpu.SemaphoreType.DMA,
    ],
)(_sc_body)


def kernel(pv_y_osgb_fourier, pv_x_osgb_fourier, pv_x_osgb, pv,
           pv_time_utc_fourier, solar_azimuth, solar_elevation,
           pv_system_row_number, embedding_table, start_idx_5_min=0):
    t = 6 + start_idx_5_min
    tf6 = lax.dynamic_slice_in_dim(pv_time_utc_fourier, t, 1, axis=1)[:, 0, :]
    az6 = lax.dynamic_slice_in_dim(solar_azimuth, t, 1, axis=1)[:, 0]
    el6 = lax.dynamic_slice_in_dim(solar_elevation, t, 1, axis=1)[:, 0]
    idx = pv_system_row_number.astype(jnp.int32)

    y5 = (pv_y_osgb_fourier.transpose(1, 2, 0).reshape(NPV, 8, BT, 128)
          .transpose(0, 2, 1, 3).reshape(NT, 8, BT, 8, 128))
    x5 = (pv_x_osgb_fourier.transpose(1, 2, 0).reshape(NPV, 8, BT, 128)
          .transpose(0, 2, 1, 3).reshape(NT, 8, BT, 8, 128))
    idx5 = (idx.transpose(1, 0).reshape(NT, 8, BT, 128)
            .transpose(0, 2, 1, 3))
    out_nat = _sc_call(y5, x5, idx5, tf6.transpose(1, 0), az6, el6,
                       embedding_table)
    return out_nat.transpose(2, 4, 1, 3, 0).reshape(B, NPV, OUTF)


# speedup vs baseline: 8.1822x; 1.0046x over previous
import functools

import jax
import jax.numpy as jnp
from jax import lax
from jax.experimental import pallas as pl
from jax.experimental.pallas import tpu as pltpu
from jax.experimental.pallas import tpu_sc as plsc

B = 4096
NPV = 200
EMB = 16
OUTF = 42
NT = NPV // 8
BT = B // 128
NGRP = NT


def _sc_body(y_hbm, x_hbm, idx_hbm, tf_hbm, az_hbm, el_hbm, tab_hbm, out_hbm,
             idx_v, emb_v, embT_v, bc_v, tf_v, az_v, el_v, yx_v,
             gsem, esem0, esem1, wsem, ysem):
    bt = lax.axis_index("s") * 2 + lax.axis_index("c")
    iota = lax.iota(jnp.int32, 16)
    boff = pl.multiple_of(bt * 128, 128)

    pltpu.sync_copy(tf_hbm.at[:, pl.ds(boff, 128)], tf_v)
    pltpu.sync_copy(az_hbm.at[pl.ds(boff, 128)], az_v)
    pltpu.sync_copy(el_hbm.at[pl.ds(boff, 128)], el_v)

    def yx_piece(p):
        f, h = p // 2, p % 2
        off, ln = (0, 13) if h == 0 else (13, 12)
        src = (y_hbm if f < 8 else x_hbm).at[pl.ds(off, ln), :, bt, f % 8, :]
        dst = out_hbm.at[f, pl.ds(off, ln), bt]
        return src, dst, ln

    rds, wds_yx = {}, {}
    s0, _, l0 = yx_piece(0)
    rds[0] = pltpu.async_copy(s0, yx_v.at[0, pl.ds(0, l0)], ysem)
    for p in range(32):
        b = p % 2
        _, dst, ln = yx_piece(p)
        rds[p].wait()
        wds_yx[p] = pltpu.async_copy(yx_v.at[b, pl.ds(0, ln)], dst, wsem)
        if p + 1 < 32:
            if p >= 1:
                wds_yx[p - 1].wait()
            sn, _, lnn = yx_piece(p + 1)
            rds[p + 1] = pltpu.async_copy(sn, yx_v.at[1 - b, pl.ds(0, lnn)],
                                          ysem)
    wds_yx[30].wait()
    wds_yx[31].wait()

    for cc in range(10):
        for i in range(8):
            if cc < 8:
                seg = tf_v[cc, pl.ds(i * 16, 16)]
            elif cc == 8:
                seg = az_v[pl.ds(i * 16, 16)]
            else:
                seg = el_v[pl.ds(i * 16, 16)]

            def brow(r, carry, _cc=cc, _i=i, _seg=seg):
                bc_v[_cc, r, pl.ds(_i * 16, 16)] = _seg
                return carry
            lax.fori_loop(0, 8, brow, 0)

    def idx_stage(j, gb):
        pltpu.sync_copy(idx_hbm.at[j, bt], idx_v.at[gb])

    def g_fire(gb):
        for j2 in range(8):
            pltpu.async_copy(tab_hbm.at[idx_v.at[gb, j2]],
                             emb_v.at[gb, j2], gsem)

    def g_drain(gb):
        for j2 in range(8):
            pltpu.make_async_copy(tab_hbm.at[idx_v.at[gb, j2]],
                                  emb_v.at[gb, j2], gsem).wait()

    def transpose(gb):
        def tr(r2, carry2):
            for e in range(EMB):
                for i in range(8):
                    vec = plsc.load_gather(
                        emb_v, [jnp.full((16,), gb, jnp.int32),
                                jnp.full((16,), r2, jnp.int32),
                                i * 16 + iota,
                                jnp.full((16,), e, jnp.int32)])
                    embT_v[gb, e, r2, pl.ds(i * 16, 16)] = vec
            return carry2
        lax.fori_loop(0, 8, tr, 0)

    esem = (esem0, esem1)

    def e_drain(gb):
        for e in range(EMB):
            pltpu.make_async_copy(embT_v.at[gb, e],
                                  out_hbm.at[26 + e, 0, bt], esem[gb]).wait()

    def sub_iter(j, gb, fire_next):
        bds = [pltpu.async_copy(bc_v.at[cc], out_hbm.at[16 + cc, j, bt], wsem)
               for cc in range(10)]
        g_drain(gb)
        if fire_next:
            idx_stage(j + 1, 1 - gb)
            g_fire(1 - gb)
        e_drain(gb)
        transpose(gb)
        for e in range(EMB):
            pltpu.async_copy(embT_v.at[gb, e], out_hbm.at[26 + e, j, bt],
                             esem[gb])
        for d in bds:
            d.wait()

    idx_stage(0, 0)
    g_fire(0)
    for gb in range(2):
        for e in range(EMB):
            pltpu.async_copy(embT_v.at[gb, e], out_hbm.at[26 + e, gb, bt],
                             esem[gb])

    def pipe(t, carry):
        sub_iter(2 * t, 0, True)
        sub_iter(2 * t + 1, 1, True)
        return carry
    lax.fori_loop(0, (NGRP - 1) // 2, pipe, 0)
    sub_iter(NGRP - 1, 0, False)
    e_drain(1)
    e_drain(0)


_sc_call = functools.partial(
    pl.kernel,
    out_type=jax.ShapeDtypeStruct((OUTF, NT, BT, 8, 128), jnp.float32),
    mesh=plsc.VectorSubcoreMesh(core_axis_name="c", subcore_axis_name="s"),
    compiler_params=pltpu.CompilerParams(
        needs_layout_passes=False, use_tc_tiling_on_sc=False),
    scratch_types=[
        pltpu.VMEM((2, 8, 128), jnp.int32),
        pltpu.VMEM((2, 8, 128, EMB), jnp.float32),
        pltpu.VMEM((2, EMB, 8, 128), jnp.float32),
        pltpu.VMEM((10, 8, 128), jnp.float32),
        pltpu.VMEM((8, 128), jnp.float32),
        pltpu.VMEM((128,), jnp.float32),
        pltpu.VMEM((128,), jnp.float32),
        pltpu.VMEM((2, 13, 8, 128), jnp.float32),
        pltpu.SemaphoreType.DMA,
        pltpu.SemaphoreType.DMA,
        pltpu.SemaphoreType.DMA,
        pltpu.SemaphoreType.DMA,
        pltpu.SemaphoreType.DMA,
    ],
)(_sc_body)


def kernel(pv_y_osgb_fourier, pv_x_osgb_fourier, pv_x_osgb, pv,
           pv_time_utc_fourier, solar_azimuth, solar_elevation,
           pv_system_row_number, embedding_table, start_idx_5_min=0):
    t = 6 + start_idx_5_min
    tf6 = lax.dynamic_slice_in_dim(pv_time_utc_fourier, t, 1, axis=1)[:, 0, :]
    az6 = lax.dynamic_slice_in_dim(solar_azimuth, t, 1, axis=1)[:, 0]
    el6 = lax.dynamic_slice_in_dim(solar_elevation, t, 1, axis=1)[:, 0]
    idx = pv_system_row_number.astype(jnp.int32)

    y5 = (pv_y_osgb_fourier.transpose(1, 2, 0).reshape(NPV, 8, BT, 128)
          .transpose(0, 2, 1, 3).reshape(NT, 8, BT, 8, 128))
    x5 = (pv_x_osgb_fourier.transpose(1, 2, 0).reshape(NPV, 8, BT, 128)
          .transpose(0, 2, 1, 3).reshape(NT, 8, BT, 8, 128))
    idx5 = (idx.transpose(1, 0).reshape(NT, 8, BT, 128)
            .transpose(0, 2, 1, 3))
    out_nat = _sc_call(y5, x5, idx5, tf6.transpose(1, 0), az6, el6,
                       embedding_table)
    return out_nat.transpose(2, 4, 1, 3, 0).reshape(B, NPV, OUTF)


# speedup vs baseline: 12.5698x; 1.5362x over previous
import functools

import jax
import jax.numpy as jnp
from jax import lax
from jax.experimental import pallas as pl
from jax.experimental.pallas import tpu as pltpu
from jax.experimental.pallas import tpu_sc as plsc

B = 4096
NPV = 200
EMB = 16
OUTF = 42
NT = NPV // 8
BT = B // 128
NGRP = NT


def _sc_body(y_hbm, x_hbm, idx_hbm, tf_hbm, az_hbm, el_hbm, tab_hbm, out_hbm,
             idx_v, emb_v, embT_v, bc_v, tf_v, az_v, el_v, yx_v,
             gsem, esem0, esem1, wsem, ysem):
    bt = lax.axis_index("s") * 2 + lax.axis_index("c")
    iota = lax.iota(jnp.int32, 16)
    boff = pl.multiple_of(bt * 128, 128)

    pltpu.sync_copy(tf_hbm.at[:, pl.ds(boff, 128)], tf_v)
    pltpu.sync_copy(az_hbm.at[pl.ds(boff, 128)], az_v)
    pltpu.sync_copy(el_hbm.at[pl.ds(boff, 128)], el_v)

    def yx_piece(p):
        f, h = p // 2, p % 2
        off, ln = (0, 13) if h == 0 else (13, 12)
        src = (y_hbm if f < 8 else x_hbm).at[pl.ds(off, ln), :, bt, f % 8, :]
        dst = out_hbm.at[f, pl.ds(off, ln), bt]
        return src, dst, ln

    rds, wds_yx = {}, {}
    s0, _, l0 = yx_piece(0)
    rds[0] = pltpu.async_copy(s0, yx_v.at[0, pl.ds(0, l0)], ysem)
    for p in range(32):
        b = p % 2
        _, dst, ln = yx_piece(p)
        rds[p].wait()
        wds_yx[p] = pltpu.async_copy(yx_v.at[b, pl.ds(0, ln)], dst, wsem)
        if p + 1 < 32:
            if p >= 1:
                wds_yx[p - 1].wait()
            sn, _, lnn = yx_piece(p + 1)
            rds[p + 1] = pltpu.async_copy(sn, yx_v.at[1 - b, pl.ds(0, lnn)],
                                          ysem)
    wds_yx[30].wait()
    wds_yx[31].wait()

    for cc in range(10):
        for i in range(8):
            if cc < 8:
                seg = tf_v[cc, pl.ds(i * 16, 16)]
            elif cc == 8:
                seg = az_v[pl.ds(i * 16, 16)]
            else:
                seg = el_v[pl.ds(i * 16, 16)]

            def brow(r, carry, _cc=cc, _i=i, _seg=seg):
                bc_v[_cc, r, pl.ds(_i * 16, 16)] = _seg
                return carry
            lax.fori_loop(0, 8, brow, 0)

    def idx_stage(j, gb):
        pltpu.sync_copy(idx_hbm.at[j, bt], idx_v.at[gb])

    def g_fire(gb):
        for j2 in range(8):
            pltpu.async_copy(tab_hbm.at[idx_v.at[gb, j2]],
                             emb_v.at[gb, j2], gsem)

    def g_drain(gb):
        for j2 in range(8):
            pltpu.make_async_copy(tab_hbm.at[idx_v.at[gb, j2]],
                                  emb_v.at[gb, j2], gsem).wait()

    def transpose(gb):
        gbs = jnp.full((16,), gb, jnp.int32)

        def tr(j2, carry2):
            j2s = jnp.full((16,), j2, jnp.int32)

            def trb(bi, carry3):
                rows = bi * 16 + iota
                for k in range(EMB):
                    diag = (iota + k) & 15
                    vec = plsc.load_gather(emb_v, [gbs, j2s, rows, diag])
                    plsc.store_scatter(embT_v, [gbs, diag, j2s, rows], vec)
                return carry3
            lax.fori_loop(0, 8, trb, 0)
            return carry2
        lax.fori_loop(0, 8, tr, 0)

    esem = (esem0, esem1)

    def e_drain(gb):
        for e in range(EMB):
            pltpu.make_async_copy(embT_v.at[gb, e],
                                  out_hbm.at[26 + e, 0, bt], esem[gb]).wait()

    def sub_iter(j, gb, fire_next):
        bds = [pltpu.async_copy(bc_v.at[cc], out_hbm.at[16 + cc, j, bt], wsem)
               for cc in range(10)]
        g_drain(gb)
        if fire_next:
            idx_stage(j + 1, 1 - gb)
            g_fire(1 - gb)
        e_drain(gb)
        transpose(gb)
        for e in range(EMB):
            pltpu.async_copy(embT_v.at[gb, e], out_hbm.at[26 + e, j, bt],
                             esem[gb])
        for d in bds:
            d.wait()

    idx_stage(0, 0)
    g_fire(0)
    for gb in range(2):
        for e in range(EMB):
            pltpu.async_copy(embT_v.at[gb, e], out_hbm.at[26 + e, gb, bt],
                             esem[gb])

    def pipe(t, carry):
        sub_iter(2 * t, 0, True)
        sub_iter(2 * t + 1, 1, True)
        return carry
    lax.fori_loop(0, (NGRP - 1) // 2, pipe, 0)
    sub_iter(NGRP - 1, 0, False)
    e_drain(1)
    e_drain(0)


_sc_call = functools.partial(
    pl.kernel,
    out_type=jax.ShapeDtypeStruct((OUTF, NT, BT, 8, 128), jnp.float32),
    mesh=plsc.VectorSubcoreMesh(core_axis_name="c", subcore_axis_name="s"),
    compiler_params=pltpu.CompilerParams(
        needs_layout_passes=False, use_tc_tiling_on_sc=False),
    scratch_types=[
        pltpu.VMEM((2, 8, 128), jnp.int32),
        pltpu.VMEM((2, 8, 128, EMB), jnp.float32),
        pltpu.VMEM((2, EMB, 8, 128), jnp.float32),
        pltpu.VMEM((10, 8, 128), jnp.float32),
        pltpu.VMEM((8, 128), jnp.float32),
        pltpu.VMEM((128,), jnp.float32),
        pltpu.VMEM((128,), jnp.float32),
        pltpu.VMEM((2, 13, 8, 128), jnp.float32),
        pltpu.SemaphoreType.DMA,
        pltpu.SemaphoreType.DMA,
        pltpu.SemaphoreType.DMA,
        pltpu.SemaphoreType.DMA,
        pltpu.SemaphoreType.DMA,
    ],
)(_sc_body)


def kernel(pv_y_osgb_fourier, pv_x_osgb_fourier, pv_x_osgb, pv,
           pv_time_utc_fourier, solar_azimuth, solar_elevation,
           pv_system_row_number, embedding_table, start_idx_5_min=0):
    t = 6 + start_idx_5_min
    tf6 = lax.dynamic_slice_in_dim(pv_time_utc_fourier, t, 1, axis=1)[:, 0, :]
    az6 = lax.dynamic_slice_in_dim(solar_azimuth, t, 1, axis=1)[:, 0]
    el6 = lax.dynamic_slice_in_dim(solar_elevation, t, 1, axis=1)[:, 0]
    idx = pv_system_row_number.astype(jnp.int32)

    y5 = (pv_y_osgb_fourier.transpose(1, 2, 0).reshape(NPV, 8, BT, 128)
          .transpose(0, 2, 1, 3).reshape(NT, 8, BT, 8, 128))
    x5 = (pv_x_osgb_fourier.transpose(1, 2, 0).reshape(NPV, 8, BT, 128)
          .transpose(0, 2, 1, 3).reshape(NT, 8, BT, 8, 128))
    idx5 = (idx.transpose(1, 0).reshape(NT, 8, BT, 128)
            .transpose(0, 2, 1, 3))
    out_nat = _sc_call(y5, x5, idx5, tf6.transpose(1, 0), az6, el6,
                       embedding_table)
    return out_nat.transpose(2, 4, 1, 3, 0).reshape(B, NPV, OUTF)


# speedup vs baseline: 12.7375x; 1.0133x over previous
import functools

import jax
import jax.numpy as jnp
from jax import lax
from jax.experimental import pallas as pl
from jax.experimental.pallas import tpu as pltpu
from jax.experimental.pallas import tpu_sc as plsc

B = 4096
NPV = 200
EMB = 16
OUTF = 42
NT = NPV // 8
BT = B // 128
NGRP = NT


def _sc_body(y_hbm, x_hbm, idx_hbm, tf_hbm, az_hbm, el_hbm, tab_hbm, out_hbm,
             idx_v, emb_v, embT_v, bc_v, tf_v, az_v, el_v, yx_v,
             gsem, esem0, esem1, wsem, ysem):
    bt = lax.axis_index("s") * 2 + lax.axis_index("c")
    iota = lax.iota(jnp.int32, 16)
    boff = pl.multiple_of(bt * 128, 128)

    pltpu.sync_copy(tf_hbm.at[:, pl.ds(boff, 128)], tf_v)
    pltpu.sync_copy(az_hbm.at[pl.ds(boff, 128)], az_v)
    pltpu.sync_copy(el_hbm.at[pl.ds(boff, 128)], el_v)

    def yx_piece(p):
        f, h = p // 2, p % 2
        off, ln = (0, 13) if h == 0 else (13, 12)
        src = (y_hbm if f < 8 else x_hbm).at[pl.ds(off, ln), :, bt, f % 8, :]
        dst = out_hbm.at[f, pl.ds(off, ln), bt]
        return src, dst, ln

    rds, wds_yx = {}, {}
    s0, _, l0 = yx_piece(0)
    rds[0] = pltpu.async_copy(s0, yx_v.at[0, pl.ds(0, l0)], ysem)
    for p in range(32):
        b = p % 2
        _, dst, ln = yx_piece(p)
        rds[p].wait()
        wds_yx[p] = pltpu.async_copy(yx_v.at[b, pl.ds(0, ln)], dst, wsem)
        if p + 1 < 32:
            if p >= 1:
                wds_yx[p - 1].wait()
            sn, _, lnn = yx_piece(p + 1)
            rds[p + 1] = pltpu.async_copy(sn, yx_v.at[1 - b, pl.ds(0, lnn)],
                                          ysem)
    wds_yx[30].wait()
    wds_yx[31].wait()

    for cc in range(10):
        for i in range(8):
            if cc < 8:
                seg = tf_v[cc, pl.ds(i * 16, 16)]
            elif cc == 8:
                seg = az_v[pl.ds(i * 16, 16)]
            else:
                seg = el_v[pl.ds(i * 16, 16)]

            def brow(r, carry, _cc=cc, _i=i, _seg=seg):
                bc_v[_cc, r, pl.ds(_i * 16, 16)] = _seg
                return carry
            lax.fori_loop(0, 8, brow, 0)

    def idx_stage(j, gb):
        pltpu.sync_copy(idx_hbm.at[j, bt], idx_v.at[gb])

    def g_fire(gb):
        for j2 in range(8):
            pltpu.async_copy(tab_hbm.at[idx_v.at[gb, j2]],
                             emb_v.at[gb, j2], gsem)

    def g_drain(gb):
        for j2 in range(8):
            pltpu.make_async_copy(tab_hbm.at[idx_v.at[gb, j2]],
                                  emb_v.at[gb, j2], gsem).wait()

    def transpose(gb):
        gbs = jnp.full((16,), gb, jnp.int32)

        def tr(j2, carry2):
            j2s = jnp.full((16,), j2, jnp.int32)

            def trb(bi, carry3):
                rows = bi * 16 + iota
                for k in range(EMB):
                    diag = (iota + k) & 15
                    vec = plsc.load_gather(emb_v, [gbs, j2s, rows, diag])
                    plsc.store_scatter(embT_v, [gbs, diag, j2s, rows], vec)
                return carry3
            lax.fori_loop(0, 8, trb, 0)
            return carry2
        lax.fori_loop(0, 8, tr, 0)

    esem = (esem0, esem1)

    def e_drain(gb):
        pltpu.make_async_copy(embT_v.at[gb],
                              out_hbm.at[pl.ds(26, EMB), 0, bt],
                              esem[gb]).wait()

    def sub_iter(j, gb, fire_next):
        bd = pltpu.async_copy(bc_v, out_hbm.at[pl.ds(16, 10), j, bt], wsem)
        g_drain(gb)
        if fire_next:
            idx_stage(j + 1, 1 - gb)
            g_fire(1 - gb)
        e_drain(gb)
        transpose(gb)
        pltpu.async_copy(embT_v.at[gb], out_hbm.at[pl.ds(26, EMB), j, bt],
                         esem[gb])
        bd.wait()

    idx_stage(0, 0)
    g_fire(0)
    for gb in range(2):
        pltpu.async_copy(embT_v.at[gb], out_hbm.at[pl.ds(26, EMB), gb, bt],
                         esem[gb])

    def pipe(t, carry):
        sub_iter(2 * t, 0, True)
        sub_iter(2 * t + 1, 1, True)
        return carry
    lax.fori_loop(0, (NGRP - 1) // 2, pipe, 0)
    sub_iter(NGRP - 1, 0, False)
    e_drain(1)
    e_drain(0)


_sc_call = functools.partial(
    pl.kernel,
    out_type=jax.ShapeDtypeStruct((OUTF, NT, BT, 8, 128), jnp.float32),
    mesh=plsc.VectorSubcoreMesh(core_axis_name="c", subcore_axis_name="s"),
    compiler_params=pltpu.CompilerParams(
        needs_layout_passes=False, use_tc_tiling_on_sc=False),
    scratch_types=[
        pltpu.VMEM((2, 8, 128), jnp.int32),
        pltpu.VMEM((2, 8, 128, EMB), jnp.float32),
        pltpu.VMEM((2, EMB, 8, 128), jnp.float32),
        pltpu.VMEM((10, 8, 128), jnp.float32),
        pltpu.VMEM((8, 128), jnp.float32),
        pltpu.VMEM((128,), jnp.float32),
        pltpu.VMEM((128,), jnp.float32),
        pltpu.VMEM((2, 13, 8, 128), jnp.float32),
        pltpu.SemaphoreType.DMA,
        pltpu.SemaphoreType.DMA,
        pltpu.SemaphoreType.DMA,
        pltpu.SemaphoreType.DMA,
        pltpu.SemaphoreType.DMA,
    ],
)(_sc_body)


def kernel(pv_y_osgb_fourier, pv_x_osgb_fourier, pv_x_osgb, pv,
           pv_time_utc_fourier, solar_azimuth, solar_elevation,
           pv_system_row_number, embedding_table, start_idx_5_min=0):
    t = 6 + start_idx_5_min
    tf6 = lax.dynamic_slice_in_dim(pv_time_utc_fourier, t, 1, axis=1)[:, 0, :]
    az6 = lax.dynamic_slice_in_dim(solar_azimuth, t, 1, axis=1)[:, 0]
    el6 = lax.dynamic_slice_in_dim(solar_elevation, t, 1, axis=1)[:, 0]
    idx = pv_system_row_number.astype(jnp.int32)

    y5 = (pv_y_osgb_fourier.transpose(1, 2, 0).reshape(NPV, 8, BT, 128)
          .transpose(0, 2, 1, 3).reshape(NT, 8, BT, 8, 128))
    x5 = (pv_x_osgb_fourier.transpose(1, 2, 0).reshape(NPV, 8, BT, 128)
          .transpose(0, 2, 1, 3).reshape(NT, 8, BT, 8, 128))
    idx5 = (idx.transpose(1, 0).reshape(NT, 8, BT, 128)
            .transpose(0, 2, 1, 3))
    out_nat = _sc_call(y5, x5, idx5, tf6.transpose(1, 0), az6, el6,
                       embedding_table)
    return out_nat.transpose(2, 4, 1, 3, 0).reshape(B, NPV, OUTF)
